# baseline (speedup 1.0000x reference)
import jax
import jax.numpy as jnp
from jax.experimental import pallas as pl
from jax.experimental.pallas import tpu as pltpu

Q = 2048
S = 4096
D = 512
NC = 8
CS = S // NC


def _matcher_kernel(q_ref, s_ref, qnorm_ref, snorm_ref, scode_ref, qlab_ref,
                    out_ref, qn_ref, rmax_ref, rcode_ref):
    c = pl.program_id(0)

    @pl.when(c == 0)
    def _():
        qn_ref[...] = q_ref[...] / qnorm_ref[...]

    sn = s_ref[...] / snorm_ref[...]

    sim = jax.lax.dot_general(
        qn_ref[...], sn, (((1,), (1,)), ((), ())),
        preferred_element_type=jnp.float32)

    bmax = jnp.max(sim, axis=1, keepdims=True)
    bcode = jnp.min(
        jnp.where(sim == bmax, scode_ref[0], jnp.int32(2 ** 30)),
        axis=1, keepdims=True)

    @pl.when(c == 0)
    def _():
        rmax_ref[...] = bmax
        rcode_ref[...] = bcode

    @pl.when(c > 0)
    def _():
        take = bmax > rmax_ref[...]
        rmax_ref[...] = jnp.where(take, bmax, rmax_ref[...])
        rcode_ref[...] = jnp.where(take, bcode, rcode_ref[...])

    @pl.when(c == NC - 1)
    def _():
        bcls = jax.lax.rem(rcode_ref[...], jnp.int32(64))
        qcls = qlab_ref[0][:, 0:1]
        out_ref[0, 0] = jnp.sum((bcls == qcls).astype(jnp.float32)) / Q


def kernel(support_features, query_features, support_labels, query_labels):
    qnorm = jnp.clip(
        jnp.linalg.norm(query_features, axis=1, keepdims=True), 1e-8)
    snorm = jnp.clip(
        jnp.linalg.norm(support_features, axis=1, keepdims=True), 1e-8)

    scls = support_labels[:, 0].astype(jnp.int32)
    scode = (jnp.arange(S, dtype=jnp.int32) * 64 + scls).reshape(1, 1, S)
    qlab = query_labels.astype(jnp.int32).reshape(1, Q, 2)

    out = pl.pallas_call(
        _matcher_kernel,
        grid=(NC,),
        in_specs=[
            pl.BlockSpec((Q, D), lambda c: (0, 0)),
            pl.BlockSpec((CS, D), lambda c: (c, 0)),
            pl.BlockSpec((Q, 1), lambda c: (0, 0)),
            pl.BlockSpec((CS, 1), lambda c: (c, 0)),
            pl.BlockSpec((1, 1, CS), lambda c: (0, 0, c)),
            pl.BlockSpec((1, Q, 2), lambda c: (0, 0, 0)),
        ],
        out_specs=pl.BlockSpec((1, 1), lambda c: (0, 0),
                               memory_space=pltpu.SMEM),
        out_shape=jax.ShapeDtypeStruct((1, 1), jnp.float32),
        scratch_shapes=[
            pltpu.VMEM((Q, D), jnp.float32),
            pltpu.VMEM((Q, 1), jnp.float32),
            pltpu.VMEM((Q, 1), jnp.int32),
        ],
        compiler_params=pltpu.CompilerParams(
            dimension_semantics=("arbitrary",),
        ),
    )(query_features, support_features, qnorm, snorm, scode, qlab)
    return out[0, 0]


# speedup vs baseline: 1.0831x; 1.0831x over previous
import jax
import jax.numpy as jnp
from jax.experimental import pallas as pl
from jax.experimental.pallas import tpu as pltpu

Q = 2048
S = 4096
D = 512
BQ = 1024
NI = Q // BQ
NC = 4
CS = S // NC


def _matcher_kernel(q_ref, s_ref, qnorm_ref, snorm_ref, scode_ref, qlab_ref,
                    out_ref, qn_ref, rmax_ref, rcode_ref):
    i = pl.program_id(0)
    c = pl.program_id(1)

    @pl.when(c == 0)
    def _():
        qn_ref[...] = q_ref[...] / qnorm_ref[...]

    sn = s_ref[...] / snorm_ref[...]

    sim = jax.lax.dot_general(
        qn_ref[...], sn, (((1,), (1,)), ((), ())),
        preferred_element_type=jnp.float32)

    bmax = jnp.max(sim, axis=1, keepdims=True)
    bcode = jnp.min(
        jnp.where(sim == bmax, scode_ref[0], jnp.int32(2 ** 30)),
        axis=1, keepdims=True)

    @pl.when(c == 0)
    def _():
        rmax_ref[...] = bmax
        rcode_ref[...] = bcode

    @pl.when(c > 0)
    def _():
        take = bmax > rmax_ref[...]
        rmax_ref[...] = jnp.where(take, bmax, rmax_ref[...])
        rcode_ref[...] = jnp.where(take, bcode, rcode_ref[...])

    @pl.when(c == NC - 1)
    def _():
        bcls = jax.lax.rem(rcode_ref[...], jnp.int32(64))
        qcls = qlab_ref[0][:, 0:1]
        cnt = jnp.sum((bcls == qcls).astype(jnp.float32))
        prev = jnp.where(i == 0, 0.0, out_ref[0, 0])
        tot = prev + cnt
        out_ref[0, 0] = jnp.where(i == NI - 1, tot / Q, tot)


def kernel(support_features, query_features, support_labels, query_labels):
    qnorm = jnp.clip(
        jnp.linalg.norm(query_features, axis=1, keepdims=True), 1e-8)
    snorm = jnp.clip(
        jnp.linalg.norm(support_features, axis=1, keepdims=True), 1e-8)

    scls = support_labels[:, 0].astype(jnp.int32)
    scode = (jnp.arange(S, dtype=jnp.int32) * 64 + scls).reshape(1, 1, S)
    qlab = query_labels.astype(jnp.int32).reshape(NI, BQ, 2)

    out = pl.pallas_call(
        _matcher_kernel,
        grid=(NI, NC),
        in_specs=[
            pl.BlockSpec((BQ, D), lambda i, c: (i, 0)),
            pl.BlockSpec((CS, D), lambda i, c: (c, 0)),
            pl.BlockSpec((BQ, 1), lambda i, c: (i, 0)),
            pl.BlockSpec((CS, 1), lambda i, c: (c, 0)),
            pl.BlockSpec((1, 1, CS), lambda i, c: (0, 0, c)),
            pl.BlockSpec((1, BQ, 2), lambda i, c: (i, 0, 0)),
        ],
        out_specs=pl.BlockSpec((1, 1), lambda i, c: (0, 0),
                               memory_space=pltpu.SMEM),
        out_shape=jax.ShapeDtypeStruct((1, 1), jnp.float32),
        scratch_shapes=[
            pltpu.VMEM((BQ, D), jnp.float32),
            pltpu.VMEM((BQ, 1), jnp.float32),
            pltpu.VMEM((BQ, 1), jnp.int32),
        ],
        compiler_params=pltpu.CompilerParams(
            dimension_semantics=("arbitrary", "arbitrary"),
        ),
    )(query_features, support_features, qnorm, snorm, scode, qlab)
    return out[0, 0]
